# baseline (speedup 1.0000x reference)
import jax
import jax.numpy as jnp
from jax.experimental import pallas as pl
from jax.experimental.pallas import tpu as pltpu


def _gemm_body(x_ref, w_ref, o_ref):
    o_ref[...] = jnp.dot(
        x_ref[...].astype(jnp.bfloat16), w_ref[...],
        preferred_element_type=jnp.float32)


def kernel(x, w_element, w_restore):
    N, Cin, H, W = x.shape
    Cout = w_restore.shape[0]
    HW = H * W
    w1 = w_element[:, :, 0, 0].astype(jnp.float32)
    w2 = w_restore[:, :, 0, 0].astype(jnp.float32)
    wfT = jnp.dot(w2, w1).T.astype(jnp.bfloat16)

    x_t = x.transpose(0, 2, 3, 1).reshape(N, HW, Cin)

    out = pl.pallas_call(
        _gemm_body,
        out_shape=jax.ShapeDtypeStruct((N, HW, Cout), jnp.float32),
        grid=(N, 2),
        in_specs=[pl.BlockSpec((None, HW // 2, Cin), lambda n, j: (n, j, 0)),
                  pl.BlockSpec((Cin, Cout), lambda n, j: (0, 0))],
        out_specs=pl.BlockSpec((None, HW // 2, Cout), lambda n, j: (n, j, 0)),
        compiler_params=pltpu.CompilerParams(
            dimension_semantics=("parallel", "parallel"),
            vmem_limit_bytes=48 << 20),
        cost_estimate=pl.CostEstimate(
            flops=2 * N * HW * Cin * Cout, transcendentals=0,
            bytes_accessed=N * HW * (Cin + Cout) * 4),
    )(x_t, wfT)
    return out.reshape(N, H, W, Cout).transpose(0, 3, 1, 2)


# speedup vs baseline: 1.1918x; 1.1918x over previous
import jax
import jax.numpy as jnp
from jax.experimental import pallas as pl
from jax.experimental.pallas import tpu as pltpu


def _gemm_body(x_ref, w_ref, o_ref):
    o_ref[...] = jnp.dot(
        x_ref[...].astype(jnp.bfloat16), w_ref[...],
        preferred_element_type=jnp.float32)


def kernel(x, w_element, w_restore):
    N, Cin, H, W = x.shape
    Cout = w_restore.shape[0]
    HW = H * W
    w1 = w_element[:, :, 0, 0].astype(jnp.float32)
    w2 = w_restore[:, :, 0, 0].astype(jnp.float32)
    wfT = jnp.dot(w2, w1).T.astype(jnp.bfloat16)

    x_t = x.transpose(0, 2, 3, 1).reshape(N // 2, 2 * HW, Cin)

    out = pl.pallas_call(
        _gemm_body,
        out_shape=jax.ShapeDtypeStruct((N // 2, 2 * HW, Cout), jnp.float32),
        grid=(N // 2,),
        in_specs=[pl.BlockSpec((None, 2 * HW, Cin), lambda n: (n, 0, 0)),
                  pl.BlockSpec((Cin, Cout), lambda n: (0, 0))],
        out_specs=pl.BlockSpec((None, 2 * HW, Cout), lambda n: (n, 0, 0)),
        compiler_params=pltpu.CompilerParams(
            dimension_semantics=("parallel",),
            vmem_limit_bytes=48 << 20),
        cost_estimate=pl.CostEstimate(
            flops=2 * N * HW * Cin * Cout, transcendentals=0,
            bytes_accessed=N * HW * (Cin + Cout) * 4),
    )(x_t, wfT)
    return out.reshape(N, H, W, Cout).transpose(0, 3, 1, 2)
